# baseline (speedup 1.0000x reference)
import functools

import jax
import jax.numpy as jnp
from jax import lax
from jax.experimental import pallas as pl
from jax.experimental.pallas import tpu as pltpu
from jax.experimental.pallas import tpu_sc as plsc

_N = 10000
_E = 320000
_D = 128
_L = 3
_HALF = _D // 2
_CHUNK = 128
_NCHUNK = _E // _CHUNK
_SUB = 16
_NSTRIPE = 640
_ZROWS = 8
_WROWS = 80
_GRP = 16
_NCHUNK_PAD = 2560
_NCHUNK_PAD = 2560
_CSUB = _NCHUNK_PAD // _SUB
_GC = 4
_NGROUP = _NCHUNK // _GC


def _pre_body(h_ref, scale_ref, bias_ref, beta_ref, f_ref, hin_ref):
    h = h_ref[...]
    mu = jnp.mean(h, axis=1, keepdims=True)
    var = jnp.mean((h - mu) ** 2, axis=1, keepdims=True)
    hn = (h - mu) * lax.rsqrt(var + 1e-5) * scale_ref[...] + bias_ref[...]
    h_in = jnp.maximum(hn, 0.0)
    m = h_in + 1e-7
    em = jnp.exp(m * beta_ref[...])
    emm = em * m
    hin_ref[...] = h_in
    f_ref[0] = jnp.concatenate([em[:, :_HALF], emm[:, :_HALF]], axis=1)
    f_ref[1] = jnp.concatenate([em[:, _HALF:], emm[:, _HALF:]], axis=1)


def _tc_pre(h, scale, bias, beta_l):
    return pl.pallas_call(
        _pre_body,
        out_shape=[jax.ShapeDtypeStruct((2, _N, _D), jnp.float32),
                   jax.ShapeDtypeStruct((_N, _D), jnp.float32)],
    )(h, scale, bias, beta_l)


def _post_body(h_ref, hin_ref, s_ref, w1_ref, b1_ref, w2_ref, b2_ref, o_ref):
    sa = s_ref[0]
    sb = s_ref[1]
    s1 = jnp.concatenate([sa[:, :_HALF], sb[:, :_HALF]], axis=1)
    s2 = jnp.concatenate([sa[:, _HALF:], sb[:, _HALF:]], axis=1)
    aggr = s2 / (s1 + 1e-16)
    u = hin_ref[...] + aggr
    t = jnp.dot(u, w1_ref[...], preferred_element_type=jnp.float32) + b1_ref[...]
    t = jnp.maximum(t, 0.0)
    z = jnp.dot(t, w2_ref[...], preferred_element_type=jnp.float32) + b2_ref[...]
    o_ref[...] = h_ref[...] + z


def _tc_post(h, h_in, s, w1, b1, w2, b2):
    return pl.pallas_call(
        _post_body,
        out_shape=jax.ShapeDtypeStruct((_N, _D), jnp.float32),
    )(h, h_in, s, w1, b1, w2, b2)


def _final_body(h_ref, scale_ref, bias_ref, w_ref, b_ref, o_ref):
    h = h_ref[...]
    mu = jnp.mean(h, axis=1, keepdims=True)
    var = jnp.mean((h - mu) ** 2, axis=1, keepdims=True)
    hn = (h - mu) * lax.rsqrt(var + 1e-5) * scale_ref[...] + bias_ref[...]
    r = jnp.maximum(hn, 0.0)
    o_ref[...] = jnp.sum(r * w_ref[...], axis=1, keepdims=True) + b_ref[...]


def _tc_final(h, scale, bias, w, b):
    return pl.pallas_call(
        _final_body,
        out_shape=jax.ShapeDtypeStruct((_N, 1), jnp.float32),
    )(h, scale, bias, w, b)


def _sc_body(f_hbm, pk_hbm, out_hbm, pkv,
             sv0, sv1, sv2, dv0, dv1, dv2,
             rows0, rows1, rows2, acc,
             gsem0, gsem1, gsem2, ssem0, ssem1, ssem2):
    srcv = [sv0, sv1, sv2]
    dstv = [dv0, dv1, dv2]
    rows = [rows0, rows1, rows2]
    gsem = [gsem0, gsem1, gsem2]
    ssem = [ssem0, ssem1, ssem2]
    c = lax.axis_index("core")
    s = lax.axis_index("subcore")

    @pl.loop(0, _ZROWS)
    def _zero_rows(r):
        for j in range(_D // 16):
            rows[0][pl.ds(r, 1), pl.ds(j * 16, 16)] = jnp.zeros((1, 16),
                                                               jnp.float32)

    @pl.loop(0, _NSTRIPE // _ZROWS)
    def _zero_acc(k):
        row = s * _NSTRIPE + k * _ZROWS

        @pl.when(row < _N)
        def _():
            pltpu.sync_copy(rows[0].at[pl.ds(0, _ZROWS)],
                            acc.at[pl.ds(row, _ZROWS)])

    plsc.subcore_barrier()

    base_node = c * _N

    @pl.loop(0, -(-_NGROUP // _SUB))
    def _edge_groups(t):
        g = t * _SUB + s

        @pl.when(g < _NGROUP)
        def _():
            pltpu.sync_copy(pk_hbm.at[pl.ds(g * _GC * _CHUNK, _GC * _CHUNK)],
                            pkv)
            for x in range(3):
                for j in range(_CHUNK // 16):
                    sl = pl.ds(x * _CHUNK + j * 16, 16)
                    v = pkv[sl]
                    dstv[x][pl.ds(j * 16, 16)] = lax.shift_right_logical(v, 14)
                    srcv[x][pl.ds(j * 16, 16)] = (v & 16383) + base_node
            gh = [None, None, None]
            sh = [None, None, None]
            gh[0] = pltpu.async_copy(f_hbm.at[srcv[0]], rows[0], gsem[0])
            gh[1] = pltpu.async_copy(f_hbm.at[srcv[1]], rows[1], gsem[1])
            gh[2] = pltpu.async_copy(f_hbm.at[srcv[2]], rows[2], gsem[2])
            for x in range(_GC):
                b = x % 3
                gh[b].wait()
                sh[b] = pltpu.async_copy(rows[b], acc.at[dstv[b]],
                                         ssem[b], add=True)
                if x + 3 < _GC:
                    sh[b].wait()
                    for j in range(_CHUNK // 16):
                        sl = pl.ds((x + 3) * _CHUNK + j * 16, 16)
                        v = pkv[sl]
                        dstv[b][pl.ds(j * 16, 16)] = lax.shift_right_logical(v, 14)
                        srcv[b][pl.ds(j * 16, 16)] = (v & 16383) + base_node
                    gh[b] = pltpu.async_copy(f_hbm.at[srcv[b]],
                                             rows[b], gsem[b])
            for b in range(3):
                sh[b].wait()

    plsc.subcore_barrier()

    @pl.loop(0, _NSTRIPE // _WROWS)
    def _writeback(k):
        row = s * _NSTRIPE + k * _WROWS

        @pl.when(row < _N)
        def _():
            pltpu.sync_copy(acc.at[pl.ds(row, _WROWS)],
                            out_hbm.at[pl.ds(base_node + row, _WROWS)])


def _sc_edge(f, pk):
    mesh = plsc.VectorSubcoreMesh(core_axis_name="core",
                                  subcore_axis_name="subcore")
    kern = functools.partial(
        pl.kernel,
        out_type=jax.ShapeDtypeStruct((2 * _N, _D), jnp.float32),
        mesh=mesh,
        scratch_types=(
            [pltpu.VMEM((_GC * _CHUNK,), jnp.int32)]
            + [pltpu.VMEM((_CHUNK,), jnp.int32)] * 6
            + [pltpu.VMEM((_CHUNK, _D), jnp.float32)] * 3
            + [pltpu.VMEM_SHARED((_N, _D), jnp.float32)]
            + [pltpu.SemaphoreType.DMA] * 6
        ),
    )(_sc_body)
    return kern(f.reshape(2 * _N, _D), pk)


def kernel(x, edge_index, ln_scale, ln_bias, W1, b1, W2, b2, beta,
           lnf_scale, lnf_bias, Wout, bout):
    src = edge_index[0].astype(jnp.int32)
    dst = edge_index[1].astype(jnp.int32)
    pk = src + (dst << 14)
    h = x
    for l in range(_L):
        f, h_in = _tc_pre(h, ln_scale[l].reshape(1, _D),
                          ln_bias[l].reshape(1, _D), beta[l].reshape(1, 1))
        s = _sc_edge(f, pk)
        h = _tc_post(h, h_in, s.reshape(2, _N, _D), W1[l],
                     b1[l].reshape(1, 2 * _D), W2[l], b2[l].reshape(1, _D))
    return _tc_final(h, lnf_scale.reshape(1, _D), lnf_bias.reshape(1, _D),
                     Wout.reshape(1, _D), bout.reshape(1, 1))


# speedup vs baseline: 1.1047x; 1.1047x over previous
import functools

import jax
import jax.numpy as jnp
from jax import lax
from jax.experimental import pallas as pl
from jax.experimental.pallas import tpu as pltpu
from jax.experimental.pallas import tpu_sc as plsc

_N = 10000
_E = 320000
_D = 128
_L = 3
_HALF = _D // 2
_CHUNK = 128
_NCHUNK = _E // _CHUNK
_SUB = 16
_NSTRIPE = 640
_ZROWS = 8
_WROWS = 80
_GRP = 16
_NCHUNK_PAD = 2560
_NCHUNK_PAD = 2560
_CSUB = _NCHUNK_PAD // _SUB
_GC = 8
_NCHUNK_P = 2504
_NGROUP = _NCHUNK_P // _GC


def _pre_body(h_ref, scale_ref, bias_ref, beta_ref, f_ref, hin_ref):
    h = h_ref[...]
    mu = jnp.mean(h, axis=1, keepdims=True)
    var = jnp.mean((h - mu) ** 2, axis=1, keepdims=True)
    hn = (h - mu) * lax.rsqrt(var + 1e-5) * scale_ref[...] + bias_ref[...]
    h_in = jnp.maximum(hn, 0.0)
    m = h_in + 1e-7
    em = jnp.exp(m * beta_ref[...])
    emm = em * m
    hin_ref[...] = h_in
    f_ref[0] = jnp.concatenate([em[:, :_HALF], emm[:, :_HALF]], axis=1)
    f_ref[1] = jnp.concatenate([em[:, _HALF:], emm[:, _HALF:]], axis=1)


def _tc_pre(h, scale, bias, beta_l):
    return pl.pallas_call(
        _pre_body,
        out_shape=[jax.ShapeDtypeStruct((2, _N, _D), jnp.float32),
                   jax.ShapeDtypeStruct((_N, _D), jnp.float32)],
    )(h, scale, bias, beta_l)


def _post_body(h_ref, hin_ref, s_ref, w1_ref, b1_ref, w2_ref, b2_ref, o_ref):
    sa = s_ref[0]
    sb = s_ref[1]
    s1 = jnp.concatenate([sa[:, :_HALF], sb[:, :_HALF]], axis=1)
    s2 = jnp.concatenate([sa[:, _HALF:], sb[:, _HALF:]], axis=1)
    aggr = s2 / (s1 + 1e-16)
    u = hin_ref[...] + aggr
    t = jnp.dot(u, w1_ref[...], preferred_element_type=jnp.float32) + b1_ref[...]
    t = jnp.maximum(t, 0.0)
    z = jnp.dot(t, w2_ref[...], preferred_element_type=jnp.float32) + b2_ref[...]
    o_ref[...] = h_ref[...] + z


def _tc_post(h, h_in, s, w1, b1, w2, b2):
    return pl.pallas_call(
        _post_body,
        out_shape=jax.ShapeDtypeStruct((_N, _D), jnp.float32),
    )(h, h_in, s, w1, b1, w2, b2)


def _final_body(h_ref, scale_ref, bias_ref, w_ref, b_ref, o_ref):
    h = h_ref[...]
    mu = jnp.mean(h, axis=1, keepdims=True)
    var = jnp.mean((h - mu) ** 2, axis=1, keepdims=True)
    hn = (h - mu) * lax.rsqrt(var + 1e-5) * scale_ref[...] + bias_ref[...]
    r = jnp.maximum(hn, 0.0)
    o_ref[...] = jnp.sum(r * w_ref[...], axis=1, keepdims=True) + b_ref[...]


def _tc_final(h, scale, bias, w, b):
    return pl.pallas_call(
        _final_body,
        out_shape=jax.ShapeDtypeStruct((_N, 1), jnp.float32),
    )(h, scale, bias, w, b)


def _sc_body(f_hbm, pk_hbm, out_hbm, pkv,
             sv0, sv1, sv2, sv3, sv4, sv5, sv6, sv7,
             dv0, dv1, dv2, dv3, dv4, dv5, dv6, dv7,
             rows0, rows1, acc, gsem0, gsem1, ssem0, ssem1):
    srcv = [sv0, sv1, sv2, sv3, sv4, sv5, sv6, sv7]
    dstv = [dv0, dv1, dv2, dv3, dv4, dv5, dv6, dv7]
    rows = [rows0, rows1]
    gsem = [gsem0, gsem1]
    ssem = [ssem0, ssem1]
    c = lax.axis_index("core")
    s = lax.axis_index("subcore")

    @pl.loop(0, _ZROWS)
    def _zero_rows(r):
        for j in range(_D // 16):
            rows[0][pl.ds(r, 1), pl.ds(j * 16, 16)] = jnp.zeros((1, 16),
                                                               jnp.float32)

    @pl.loop(0, _NSTRIPE // _ZROWS)
    def _zero_acc(k):
        row = s * _NSTRIPE + k * _ZROWS

        @pl.when(row < _N)
        def _():
            pltpu.sync_copy(rows[0].at[pl.ds(0, _ZROWS)],
                            acc.at[pl.ds(row, _ZROWS)])

    @pl.when(s == 0)
    def _zero_dump():
        pltpu.sync_copy(rows[0].at[pl.ds(0, 8)], acc.at[pl.ds(_N, 8)])

    plsc.subcore_barrier()

    base_node = c * _N

    @pl.loop(0, -(-_NGROUP // _SUB))
    def _edge_groups(t):
        g = t * _SUB + s

        @pl.when(g < _NGROUP)
        def _():
            pltpu.sync_copy(pk_hbm.at[pl.ds(g * _GC * _CHUNK, _GC * _CHUNK)],
                            pkv)
            for x in range(_GC):
                for j in range(_CHUNK // 16):
                    sl = pl.ds(x * _CHUNK + j * 16, 16)
                    v = pkv[sl]
                    dstv[x][pl.ds(j * 16, 16)] = lax.shift_right_logical(v, 14)
                    srcv[x][pl.ds(j * 16, 16)] = (v & 16383) + base_node
            gh = [None, None]
            sh = [None, None]
            gh[0] = pltpu.async_copy(f_hbm.at[srcv[0]], rows[0], gsem[0])
            gh[1] = pltpu.async_copy(f_hbm.at[srcv[1]], rows[1], gsem[1])
            for x in range(_GC):
                b = x & 1
                gh[b].wait()
                sh[b] = pltpu.async_copy(rows[b], acc.at[dstv[x]],
                                         ssem[b], add=True)
                if x + 2 < _GC:
                    sh[b].wait()
                    gh[b] = pltpu.async_copy(f_hbm.at[srcv[x + 2]],
                                             rows[b], gsem[b])
            sh[0].wait()
            sh[1].wait()

    plsc.subcore_barrier()

    @pl.loop(0, _NSTRIPE // _WROWS)
    def _writeback(k):
        row = s * _NSTRIPE + k * _WROWS

        @pl.when(row < _N)
        def _():
            pltpu.sync_copy(acc.at[pl.ds(row, _WROWS)],
                            out_hbm.at[pl.ds(base_node + row, _WROWS)])


def _sc_edge(f, pk):
    mesh = plsc.VectorSubcoreMesh(core_axis_name="core",
                                  subcore_axis_name="subcore")
    kern = functools.partial(
        pl.kernel,
        out_type=jax.ShapeDtypeStruct((2 * _N, _D), jnp.float32),
        mesh=mesh,
        scratch_types=(
            [pltpu.VMEM((_GC * _CHUNK,), jnp.int32)]
            + [pltpu.VMEM((_CHUNK,), jnp.int32)] * 16
            + [pltpu.VMEM((_CHUNK, _D), jnp.float32)] * 2
            + [pltpu.VMEM_SHARED((_N + 8, _D), jnp.float32)]
            + [pltpu.SemaphoreType.DMA] * 4
        ),
    )(_sc_body)
    return kern(f.reshape(2 * _N, _D), pk)


def kernel(x, edge_index, ln_scale, ln_bias, W1, b1, W2, b2, beta,
           lnf_scale, lnf_bias, Wout, bout):
    src = edge_index[0].astype(jnp.int32)
    dst = edge_index[1].astype(jnp.int32)
    pk = src + (dst << 14)
    pk = jnp.pad(pk, ((0, (_NCHUNK_P - _NCHUNK) * _CHUNK),),
                 constant_values=_N << 14)
    h = x
    for l in range(_L):
        f, h_in = _tc_pre(h, ln_scale[l].reshape(1, _D),
                          ln_bias[l].reshape(1, _D), beta[l].reshape(1, 1))
        s = _sc_edge(f, pk)
        h = _tc_post(h, h_in, s.reshape(2, _N, _D), W1[l],
                     b1[l].reshape(1, 2 * _D), W2[l], b2[l].reshape(1, _D))
    return _tc_final(h, lnf_scale.reshape(1, _D), lnf_bias.reshape(1, _D),
                     Wout.reshape(1, _D), bout.reshape(1, 1))
